# Initial kernel scaffold; baseline (speedup 1.0000x reference)
import jax
import jax.numpy as jnp
from jax.experimental import pallas as pl


def kernel(user_emb, item_emb, edge_index, edge_values, W1, b1, W2, b2):
    raise NotImplementedError("write your pallas kernel here")


# speedup vs baseline: 1.3544x; 1.3544x over previous
import functools

import jax
import jax.numpy as jnp
from jax import lax
from jax.experimental import pallas as pl
from jax.experimental.pallas import tpu as pltpu
from jax.experimental.pallas import tpu_sc as plsc

EMB = 128
MLP = 64
HALF = 160000
B_TEMP = 0.5

NC = 2
NS = 16
NW = NC * NS
E = 128
PAD_HALF = 163840
CH = PAD_HALF // NW
NR = CH // E


def _proj_body(ne, w1a, w1b, b1r, o1, o2):
    x = ne[...]
    o1[...] = jax.lax.dot_general(
        x, w1a[...], (((1,), (0,)), ((), ())),
        preferred_element_type=jnp.float32,
        precision=jax.lax.Precision.HIGHEST) + b1r[...]
    o2[...] = jax.lax.dot_general(
        x, w1b[...], (((1,), (0,)), ((), ())),
        preferred_element_type=jnp.float32,
        precision=jax.lax.Precision.HIGHEST)


def _proj(node_emb, w1a, w1b, b1r):
    rows = node_emb.shape[0]
    blk = 1000
    return pl.pallas_call(
        _proj_body,
        grid=(rows // blk,),
        in_specs=[
            pl.BlockSpec((blk, EMB), lambda i: (i, 0)),
            pl.BlockSpec((EMB, MLP), lambda i: (0, 0)),
            pl.BlockSpec((EMB, MLP), lambda i: (0, 0)),
            pl.BlockSpec((1, MLP), lambda i: (0, 0)),
        ],
        out_specs=[
            pl.BlockSpec((blk, MLP), lambda i: (i, 0)),
            pl.BlockSpec((blk, MLP), lambda i: (i, 0)),
        ],
        out_shape=[
            jax.ShapeDtypeStruct((rows, MLP), jnp.float32),
            jax.ShapeDtypeStruct((rows, MLP), jnp.float32),
        ],
    )(node_emb, w1a, w1b, b1r)


_MESH = plsc.VectorSubcoreMesh(
    core_axis_name="c", subcore_axis_name="s", num_cores=NC, num_subcores=NS)


@functools.partial(
    pl.kernel,
    out_type=(
        jax.ShapeDtypeStruct((PAD_HALF,), jnp.float32),
        jax.ShapeDtypeStruct((NW, 16), jnp.float32),
    ),
    mesh=_MESH,
    compiler_params=pltpu.CompilerParams(
        needs_layout_passes=False, use_tc_tiling_on_sc=False),
    scratch_types=[
        pltpu.VMEM((E,), jnp.int32),
        pltpu.VMEM((E,), jnp.int32),
        pltpu.VMEM((E, MLP), jnp.float32),
        pltpu.VMEM((E, MLP), jnp.float32),
        pltpu.VMEM((E,), jnp.float32),
        pltpu.VMEM((E,), jnp.float32),
        pltpu.VMEM((E,), jnp.float32),
        pltpu.VMEM((MLP, 16), jnp.float32),
        pltpu.VMEM((16,), jnp.float32),
        pltpu.VMEM((16,), jnp.float32),
        pltpu.SemaphoreType.DMA,
        pltpu.SemaphoreType.DMA,
    ],
)
def _edge(p1, p2, src, dst, noise, ev, w2, b2,
          vals_out, psum_out,
          isrc, idst, rows1, rows2, nbuf, evbuf, obuf,
          w2v, b2v, asum, sem1, sem2):
    wid = lax.axis_index("s") * NC + lax.axis_index("c")
    base = wid * CH

    pltpu.sync_copy(w2, w2v)
    pltpu.sync_copy(b2, b2v)
    b2r = b2v[...]
    asum[...] = jnp.zeros((16,), jnp.float32)

    def round_body(r, carry):
        rbase = base + r * E
        pltpu.sync_copy(src.at[pl.ds(rbase, E)], isrc)
        pltpu.sync_copy(dst.at[pl.ds(rbase, E)], idst)
        pltpu.sync_copy(noise.at[pl.ds(rbase, E)], nbuf)
        pltpu.sync_copy(ev.at[pl.ds(rbase, E)], evbuf)
        c1 = pltpu.async_copy(p1.at[isrc], rows1, sem1)
        c2 = pltpu.async_copy(p2.at[idst], rows2, sem2)
        c1.wait()
        c2.wait()

        def group_body(g, c):
            eids = lax.iota(jnp.int32, 16) + 16 * g
            accs = [jnp.zeros((16,), jnp.float32) for _ in range(4)]
            for k in range(MLP):
                kv = jnp.full((16,), k, jnp.int32)
                a = plsc.load_gather(rows1, [eids, kv])
                b = plsc.load_gather(rows2, [eids, kv])
                accs[k % 4] = accs[k % 4] + jnp.maximum(a + b, 0.0) * w2v[k]
            logit = (accs[0] + accs[1]) + (accs[2] + accs[3])
            sl = pl.ds(g * 16, 16)
            x = (logit + b2r + nbuf[sl]) * (1.0 / B_TEMP)
            s = 1.0 / (1.0 + jnp.exp(-x))
            obuf[sl] = evbuf[sl] * s
            asum[...] = asum[...] + s
            return c

        lax.fori_loop(0, E // 16, group_body, 0)
        pltpu.sync_copy(obuf, vals_out.at[pl.ds(rbase, E)])
        return carry

    lax.fori_loop(0, NR, round_body, 0)
    pltpu.sync_copy(asum, psum_out.at[wid])


def kernel(user_emb, item_emb, edge_index, edge_values, W1, b1, W2, b2):
    node_emb = jnp.concatenate([user_emb, item_emb], axis=0)
    src = edge_index[0, :HALF]
    dst = edge_index[1, :HALF]

    p1, p2 = _proj(node_emb, W1[:EMB], W1[EMB:], b1.reshape(1, MLP))

    bias = 0.0 + 0.0001
    eps_key = jax.random.key(42)
    u = jax.random.uniform(eps_key, (HALF, 1), dtype=jnp.float32)
    eps = (bias - (1.0 - bias)) * u + (1.0 - bias)
    noise = (jnp.log(eps) - jnp.log(1.0 - eps)).reshape(HALF)

    pad = PAD_HALF - HALF
    src_p = jnp.concatenate([src, jnp.zeros((pad,), jnp.int32)])
    dst_p = jnp.concatenate([dst, jnp.zeros((pad,), jnp.int32)])
    ev_p = jnp.concatenate([edge_values[:HALF], jnp.zeros((pad,), jnp.float32)])
    noise_p = jnp.concatenate([noise, jnp.full((pad,), -1e8, jnp.float32)])

    w2s = jnp.broadcast_to(W2.reshape(MLP, 1), (MLP, 16))
    vals_p, psums = _edge(p1, p2, src_p, dst_p, noise_p, ev_p,
                          w2s, jnp.broadcast_to(b2, (16,)))

    new_vals = vals_p[:HALF]
    mean_edge_weight = jnp.sum(psums) / HALF
    sym_vals = jnp.concatenate([new_vals, new_vals])
    sym_rows = jnp.concatenate([src, dst])
    sym_cols = jnp.concatenate([dst, src])
    return sym_vals, sym_rows, sym_cols, mean_edge_weight
